# Initial kernel scaffold; baseline (speedup 1.0000x reference)
import jax
import jax.numpy as jnp
from jax.experimental import pallas as pl


def kernel(x, table):
    raise NotImplementedError("write your pallas kernel here")


# speedup vs baseline: 8.4594x; 8.4594x over previous
import functools

import jax
import jax.numpy as jnp
from jax import lax
from jax.experimental import pallas as pl
from jax.experimental.pallas import tpu as pltpu
from jax.experimental.pallas import tpu_sc as plsc

_NTOKEN = 100000
_D = 128
_B = 4096 * 200

_info = plsc.get_sparse_core_info()
_NC, _NS = _info.num_cores, _info.num_subcores
_NW = _NC * _NS
_BPW = _B // _NW
_CH = 128
_NCHUNK = _BPW // _CH
_NBUF = 4

_mesh = plsc.VectorSubcoreMesh(core_axis_name="c", subcore_axis_name="s")


@functools.partial(
    pl.kernel,
    mesh=_mesh,
    out_type=jax.ShapeDtypeStruct((_B, _D), jnp.float32),
    scratch_types=[
        pltpu.VMEM((_BPW,), jnp.int32),
        pltpu.VMEM((_NBUF, _CH, _D), jnp.float32),
        pltpu.SemaphoreType.DMA,
        pltpu.SemaphoreType.DMA,
        pltpu.SemaphoreType.DMA,
        pltpu.SemaphoreType.DMA,
        pltpu.SemaphoreType.DMA,
        pltpu.SemaphoreType.DMA,
        pltpu.SemaphoreType.DMA,
        pltpu.SemaphoreType.DMA,
    ],
)
def _emb_lookup(x_hbm, table_hbm, out_hbm, idx_v, rows_v,
                g0, g1, g2, g3, s0, s1, s2, s3):
    gsem = (g0, g1, g2, g3)
    ssem = (s0, s1, s2, s3)
    wid = lax.axis_index("s") * _NC + lax.axis_index("c")
    base = wid * _BPW

    pltpu.sync_copy(x_hbm.at[pl.ds(base, _BPW)], idx_v)

    def gcopy(ci, b):
        return pltpu.make_async_copy(
            table_hbm.at[idx_v.at[pl.ds(ci * _CH, _CH)]],
            rows_v.at[b],
            gsem[b],
        )

    def scopy(ci, b):
        return pltpu.make_async_copy(
            rows_v.at[b],
            out_hbm.at[pl.ds(base + ci * _CH, _CH)],
            ssem[b],
        )

    for b in range(_NBUF):
        gcopy(b, b).start()

    def body(k, carry):
        for b in range(_NBUF):
            ci = _NBUF * k + b
            gcopy(ci, b).wait()
            scopy(ci, b).start()
            scopy(ci, b).wait()
            gcopy(ci + _NBUF, b).start()
        return carry

    lax.fori_loop(0, _NCHUNK // _NBUF - 1, body, 0)

    for b in range(_NBUF):
        ci = _NCHUNK - _NBUF + b
        gcopy(ci, b).wait()
        scopy(ci, b).start()
        scopy(ci, b).wait()


def kernel(x, table):
    x_flat = x.reshape(-1).astype(jnp.int32)
    table_eff = table.at[_NTOKEN].set(0.0)
    out = _emb_lookup(x_flat, table_eff)
    return out.reshape(x.shape + (_D,))


# speedup vs baseline: 9.2724x; 1.0961x over previous
import functools

import jax
import jax.numpy as jnp
from jax import lax
from jax.experimental import pallas as pl
from jax.experimental.pallas import tpu as pltpu
from jax.experimental.pallas import tpu_sc as plsc

_NTOKEN = 100000
_D = 128
_B = 4096 * 200

_info = plsc.get_sparse_core_info()
_NC, _NS = _info.num_cores, _info.num_subcores
_NW = _NC * _NS
_BPW = _B // _NW
_CH = 128
_NCHUNK = _BPW // _CH
_NBUF = 4
_L = 16

_mesh = plsc.VectorSubcoreMesh(core_axis_name="c", subcore_axis_name="s")


@functools.partial(
    pl.kernel,
    mesh=_mesh,
    compiler_params=pltpu.CompilerParams(needs_layout_passes=False),
    out_type=jax.ShapeDtypeStruct((_B, _D), jnp.float32),
    scratch_types=[
        pltpu.VMEM((_BPW,), jnp.int32),
        pltpu.VMEM((_NBUF, _CH, _D), jnp.float32),
        pltpu.SemaphoreType.DMA,
        pltpu.SemaphoreType.DMA,
        pltpu.SemaphoreType.DMA,
        pltpu.SemaphoreType.DMA,
        pltpu.SemaphoreType.DMA,
        pltpu.SemaphoreType.DMA,
        pltpu.SemaphoreType.DMA,
        pltpu.SemaphoreType.DMA,
    ],
)
def _emb_lookup(x_hbm, table_hbm, out_hbm, idx_v, rows_v,
                g0, g1, g2, g3, s0, s1, s2, s3):
    gsem = (g0, g1, g2, g3)
    ssem = (s0, s1, s2, s3)
    wid = lax.axis_index("s") * _NC + lax.axis_index("c")
    base = wid * _BPW

    pltpu.sync_copy(x_hbm.at[pl.ds(base, _BPW)], idx_v)

    def gcopy(ci, b):
        return pltpu.make_async_copy(
            table_hbm.at[idx_v.at[pl.ds(ci * _CH, _CH)]],
            rows_v.at[b],
            gsem[b],
        )

    def scopy(ci, b):
        return pltpu.make_async_copy(
            rows_v.at[b],
            out_hbm.at[pl.ds(base + ci * _CH, _CH)],
            ssem[b],
        )

    def fixup(ci, b):
        def chunk_max(g, acc):
            idx_g = idx_v[pl.ds(ci * _CH + g * _L, _L)]
            return jnp.maximum(acc, idx_g)

        mx = lax.fori_loop(
            0, _CH // _L, chunk_max, jnp.zeros((_L,), jnp.int32))
        any_pad = jnp.any(mx >= _NTOKEN)

        @pl.when(any_pad)
        def _():
            zeros = jnp.zeros((_L,), jnp.float32)

            def grp(g, carry):
                idx_g = idx_v[pl.ds(ci * _CH + g * _L, _L)]
                m = idx_g == _NTOKEN

                @pl.when(jnp.any(m))
                def _():
                    rowpos = g * _L + jnp.arange(_L, dtype=jnp.int32)
                    for j in range(_D):
                        colpos = jnp.full((_L,), j, jnp.int32)
                        plsc.store_scatter(
                            rows_v.at[b], [rowpos, colpos], zeros, mask=m)
                return carry

            lax.fori_loop(0, _CH // _L, grp, 0)

    for b in range(_NBUF):
        gcopy(b, b).start()

    def body(k, carry):
        for b in range(_NBUF):
            ci = _NBUF * k + b
            gcopy(ci, b).wait()
            fixup(ci, b)
            scopy(ci, b).start()
            scopy(ci, b).wait()

            @pl.when(ci + _NBUF < _NCHUNK)
            def _():
                gcopy(ci + _NBUF, b).start()
        return carry

    lax.fori_loop(0, _NCHUNK // _NBUF, body, 0)


def kernel(x, table):
    x_flat = x.reshape(-1).astype(jnp.int32)
    out = _emb_lookup(x_flat, table)
    return out.reshape(x.shape + (_D,))
